# Initial kernel scaffold; baseline (speedup 1.0000x reference)
import functools

import jax
import jax.numpy as jnp
from jax import lax
from jax.experimental import pallas as pl
from jax.experimental.pallas import tpu as pltpu
from jax.experimental.pallas import tpu_sc as plsc

N = 10000
E = 320000
NC, NS = 2, 16
NW = NC * NS
CHUNK = 80
ROW_T = 640
ROW_LAST = N - (NS - 1) * ROW_T

_f32 = jnp.float32
_i32 = jnp.int32


def _mesh():
    return plsc.VectorSubcoreMesh(core_axis_name="c", subcore_axis_name="s")


def _zero_rows_2d(zb, d):
    z16 = jnp.zeros((16,), _f32)

    def body(i, _):
        for j in range(d // 16):
            zb[i, pl.ds(j * 16, 16)] = z16
        return 0

    lax.fori_loop(0, 80, body, 0)


def _zero_1d(ref, n):
    z16 = jnp.zeros((16,), _f32)

    def body(i, _):
        ref[pl.ds(i * 16, 16)] = z16
        return 0

    lax.fori_loop(0, n // 16, body, 0)


def _rsqrt16(d):
    i = plsc.bitcast(d, _i32)
    i = 0x5F3759DF - lax.shift_right_logical(i, 1)
    y = plsc.bitcast(i, _f32)
    for _ in range(3):
        y = y * (1.5 - 0.5 * d * y * y)
    return y


def _degree_call(src, dst):
    EPT = E // NW
    NCH = EPT // CHUNK

    @functools.partial(
        pl.kernel,
        mesh=_mesh(),
        out_type=[jax.ShapeDtypeStruct((N,), _f32)] * 4,
        scratch_types=[
            pltpu.VMEM((CHUNK,), _i32),
            pltpu.VMEM((CHUNK,), _f32),
            pltpu.VMEM((ROW_T,), _f32),
            pltpu.VMEM_SHARED((N,), _f32),
            pltpu.VMEM_SHARED((N,), _f32),
        ],
    )
    def deg_k(src_hbm, dst_hbm, do0, di0, do1, di1, idx_v, ones_v, zb,
              acc_o, acc_i):
        cc = lax.axis_index("c")
        ss = lax.axis_index("s")
        one16 = jnp.ones((16,), _f32)
        for g in range(CHUNK // 16):
            ones_v[pl.ds(g * 16, 16)] = one16
        _zero_1d(zb, ROW_T)
        r0 = ss * ROW_T

        @pl.when(ss < NS - 1)
        def _():
            pltpu.sync_copy(zb, acc_o.at[pl.ds(r0, ROW_T)])
            pltpu.sync_copy(zb, acc_i.at[pl.ds(r0, ROW_T)])

        @pl.when(ss == NS - 1)
        def _():
            pltpu.sync_copy(zb.at[pl.ds(0, ROW_LAST)],
                            acc_o.at[pl.ds(r0, ROW_LAST)])
            pltpu.sync_copy(zb.at[pl.ds(0, ROW_LAST)],
                            acc_i.at[pl.ds(r0, ROW_LAST)])

        plsc.subcore_barrier()

        e0 = (cc * NS + ss) * EPT

        def chunk(k, _):
            base = e0 + k * CHUNK
            pltpu.sync_copy(src_hbm.at[pl.ds(base, CHUNK)], idx_v)
            pltpu.sync_copy(ones_v, acc_o.at[idx_v], add=True)
            pltpu.sync_copy(dst_hbm.at[pl.ds(base, CHUNK)], idx_v)
            pltpu.sync_copy(ones_v, acc_i.at[idx_v], add=True)
            return 0

        lax.fori_loop(0, NCH, chunk, 0)
        plsc.subcore_barrier()

        def wout(o_ref, i_ref, off, size):
            seg = pl.ds(off, size)
            pltpu.sync_copy(acc_o.at[seg], o_ref.at[seg])
            pltpu.sync_copy(acc_i.at[seg], i_ref.at[seg])

        @pl.when(jnp.logical_and(cc == 0, ss < NS - 1))
        def _():
            wout(do0, di0, r0, ROW_T)

        @pl.when(jnp.logical_and(cc == 0, ss == NS - 1))
        def _():
            wout(do0, di0, r0, ROW_LAST)

        @pl.when(jnp.logical_and(cc == 1, ss < NS - 1))
        def _():
            wout(do1, di1, r0, ROW_T)

        @pl.when(jnp.logical_and(cc == 1, ss == NS - 1))
        def _():
            wout(do1, di1, r0, ROW_LAST)

    return deg_k(src, dst)


def _coef_call(do0, di0, do1, di1, src, dst, ew):
    EPT = E // NW
    NCH = EPT // CHUNK

    @functools.partial(
        pl.kernel,
        mesh=_mesh(),
        out_type=jax.ShapeDtypeStruct((E,), _f32),
        scratch_types=[
            pltpu.VMEM((N,), _f32),
            pltpu.VMEM((N,), _f32),
            pltpu.VMEM((N,), _f32),
            pltpu.VMEM((N,), _f32),
            pltpu.VMEM((CHUNK,), _i32),
            pltpu.VMEM((CHUNK,), _i32),
            pltpu.VMEM((CHUNK,), _f32),
            pltpu.VMEM((CHUNK,), _f32),
        ],
    )
    def coef_k(do0_h, di0_h, do1_h, di1_h, src_h, dst_h, ew_h, c_h,
               pa, pb, no_v, ni_v, srcv, dstv, ewv, cv):
        cc = lax.axis_index("c")
        ss = lax.axis_index("s")
        pltpu.sync_copy(do0_h, pa)
        pltpu.sync_copy(do1_h, pb)

        def nbody(i, _):
            sl = pl.ds(i * 16, 16)
            no_v[sl] = _rsqrt16(jnp.maximum(pa[sl] + pb[sl], 1.0))
            return 0

        lax.fori_loop(0, N // 16, nbody, 0)
        pltpu.sync_copy(di0_h, pa)
        pltpu.sync_copy(di1_h, pb)

        def nbody2(i, _):
            sl = pl.ds(i * 16, 16)
            ni_v[sl] = _rsqrt16(jnp.maximum(pa[sl] + pb[sl], 1.0))
            return 0

        lax.fori_loop(0, N // 16, nbody2, 0)

        e0 = (cc * NS + ss) * EPT

        def chunk(k, _):
            base = e0 + k * CHUNK
            pltpu.sync_copy(src_h.at[pl.ds(base, CHUNK)], srcv)
            pltpu.sync_copy(dst_h.at[pl.ds(base, CHUNK)], dstv)
            pltpu.sync_copy(ew_h.at[pl.ds(base, CHUNK)], ewv)
            for g in range(CHUNK // 16):
                sl = pl.ds(g * 16, 16)
                no16 = plsc.load_gather(no_v, [srcv[sl]])
                ni16 = plsc.load_gather(ni_v, [dstv[sl]])
                cv[sl] = ewv[sl] * no16 * ni16
            pltpu.sync_copy(cv, c_h.at[pl.ds(base, CHUNK)])
            return 0

        lax.fori_loop(0, NCH, chunk, 0)

    return coef_k(do0, di0, do1, di1, src, dst, ew)


def _msgpass_call(h_args, src, dst, c, d, col_split):
    EPT = E // NS if col_split else E // NW
    NCH = EPT // CHUNK

    @functools.partial(
        pl.kernel,
        mesh=_mesh(),
        out_type=[jax.ShapeDtypeStruct((N, d), _f32)] * 2,
        scratch_types=[
            pltpu.VMEM((CHUNK,), _i32),
            pltpu.VMEM((CHUNK,), _i32),
            pltpu.VMEM((CHUNK,), _f32),
            pltpu.VMEM((CHUNK, d), _f32),
            pltpu.VMEM((80, d), _f32),
            pltpu.VMEM_SHARED((N, d), _f32),
            pltpu.SemaphoreType.DMA,
        ],
    )
    def mp_k(*refs):
        if col_split:
            ha, hb, src_h, dst_h, c_h = refs[:5]
            outs = refs[5:7]
            srcv, dstv, cv, rows, zb, acc, sem = refs[7:]
        else:
            h_h, src_h, dst_h, c_h = refs[:4]
            outs = refs[4:6]
            srcv, dstv, cv, rows, zb, acc, sem = refs[6:]
        cc = lax.axis_index("c")
        ss = lax.axis_index("s")

        _zero_rows_2d(zb, d)
        r0 = ss * ROW_T

        def zero_acc(nseg):
            def zbody(k, _):
                pltpu.sync_copy(zb, acc.at[pl.ds(r0 + k * 80, 80)])
                return 0
            lax.fori_loop(0, nseg, zbody, 0)

        @pl.when(ss < NS - 1)
        def _():
            zero_acc(ROW_T // 80)

        @pl.when(ss == NS - 1)
        def _():
            zero_acc(ROW_LAST // 80)

        plsc.subcore_barrier()

        e0 = (ss if col_split else cc * NS + ss) * EPT

        def edge_loop(h_ref):
            def chunk(k, _):
                base = e0 + k * CHUNK
                pltpu.sync_copy(src_h.at[pl.ds(base, CHUNK)], srcv)
                pltpu.sync_copy(dst_h.at[pl.ds(base, CHUNK)], dstv)
                pltpu.sync_copy(c_h.at[pl.ds(base, CHUNK)], cv)
                pltpu.async_copy(h_ref.at[srcv], rows, sem).wait()

                def row(i, _):
                    ci = plsc.load_gather(cv, [jnp.broadcast_to(i, (16,))])
                    for j in range(d // 16):
                        sl = pl.ds(j * 16, 16)
                        rows[i, sl] = rows[i, sl] * ci
                    return 0

                lax.fori_loop(0, CHUNK, row, 0)
                pltpu.sync_copy(rows, acc.at[dstv], add=True)
                return 0

            lax.fori_loop(0, NCH, chunk, 0)

        if col_split:
            @pl.when(cc == 0)
            def _():
                edge_loop(ha)

            @pl.when(cc == 1)
            def _():
                edge_loop(hb)
        else:
            edge_loop(h_h)

        plsc.subcore_barrier()

        def wout(o_ref, off, size):
            seg = pl.ds(off, size)
            pltpu.sync_copy(acc.at[seg], o_ref.at[seg])

        @pl.when(jnp.logical_and(cc == 0, ss < NS - 1))
        def _():
            wout(outs[0], r0, ROW_T)

        @pl.when(jnp.logical_and(cc == 0, ss == NS - 1))
        def _():
            wout(outs[0], r0, ROW_LAST)

        @pl.when(jnp.logical_and(cc == 1, ss < NS - 1))
        def _():
            wout(outs[1], r0, ROW_T)

        @pl.when(jnp.logical_and(cc == 1, ss == NS - 1))
        def _():
            wout(outs[1], r0, ROW_LAST)

    return mp_k(*h_args, src, dst, c)


_BLK = 400


def _tc_l1(p0, p1, W1, b1):
    def body(p0_r, p1_r, w_r, b_r, oa_r, ob_r):
        a = p0_r[...] + p1_r[...]
        h = jnp.dot(a, w_r[...], preferred_element_type=_f32) + b_r[...]
        h = jnp.maximum(h, 0.0)
        oa_r[...] = h[:, :128]
        ob_r[...] = h[:, 128:]

    return pl.pallas_call(
        body,
        grid=(N // _BLK,),
        in_specs=[
            pl.BlockSpec((_BLK, 128), lambda r: (r, 0)),
            pl.BlockSpec((_BLK, 128), lambda r: (r, 0)),
            pl.BlockSpec((128, 256), lambda r: (0, 0)),
            pl.BlockSpec((1, 256), lambda r: (0, 0)),
        ],
        out_specs=[pl.BlockSpec((_BLK, 128), lambda r: (r, 0))] * 2,
        out_shape=[jax.ShapeDtypeStruct((N, 128), _f32)] * 2,
    )(p0, p1, W1, b1)


def _tc_l2l3(aa, ab, W2, b2, W3):
    def body(aa_r, ab_r, w2_r, b2_r, w3_r, o_r):
        h = (jnp.dot(aa_r[...], w2_r[:128, :], preferred_element_type=_f32)
             + jnp.dot(ab_r[...], w2_r[128:, :], preferred_element_type=_f32)
             + b2_r[...])
        h = jnp.maximum(h, 0.0)
        o_r[...] = jnp.dot(h, w3_r[...], preferred_element_type=_f32)

    return pl.pallas_call(
        body,
        grid=(N // _BLK,),
        in_specs=[
            pl.BlockSpec((_BLK, 128), lambda r: (r, 0)),
            pl.BlockSpec((_BLK, 128), lambda r: (r, 0)),
            pl.BlockSpec((256, 256), lambda r: (0, 0)),
            pl.BlockSpec((1, 256), lambda r: (0, 0)),
            pl.BlockSpec((256, 64), lambda r: (0, 0)),
        ],
        out_specs=pl.BlockSpec((_BLK, 64), lambda r: (r, 0)),
        out_shape=jax.ShapeDtypeStruct((N, 64), _f32),
    )(aa, ab, W2, b2, W3)


def _tc_out(p0, p1, b3):
    def body(p0_r, p1_r, b_r, o_r):
        z = p0_r[...] + p1_r[...] + b_r[...]
        m = jnp.max(z, axis=-1, keepdims=True)
        e = jnp.exp(z - m)
        o_r[...] = e / jnp.sum(e, axis=-1, keepdims=True)

    return pl.pallas_call(
        body,
        grid=(N // _BLK,),
        in_specs=[
            pl.BlockSpec((_BLK, 64), lambda r: (r, 0)),
            pl.BlockSpec((_BLK, 64), lambda r: (r, 0)),
            pl.BlockSpec((1, 64), lambda r: (0, 0)),
        ],
        out_specs=pl.BlockSpec((_BLK, 64), lambda r: (r, 0)),
        out_shape=jax.ShapeDtypeStruct((N, 64), _f32),
    )(p0, p1, b3)


def kernel(features, edge_index, edge_weight, W1, b1, W2, b2, W3, b3):
    src = edge_index[0]
    dst = edge_index[1]
    do0, di0, do1, di1 = _degree_call(src, dst)
    c = _coef_call(do0, di0, do1, di1, src, dst, edge_weight)
    p1a, p1b = _msgpass_call((features,), src, dst, c, 128, col_split=False)
    h1a, h1b = _tc_l1(p1a, p1b, W1, b1.reshape(1, -1))
    a2a, a2b = _msgpass_call((h1a, h1b), src, dst, c, 128, col_split=True)
    t = _tc_l2l3(a2a, a2b, W2, b2.reshape(1, -1), W3)
    p3a, p3b = _msgpass_call((t,), src, dst, c, 64, col_split=False)
    return _tc_out(p3a, p3b, b3.reshape(1, -1))


# speedup vs baseline: 6.1986x; 6.1986x over previous
import functools

import jax
import jax.numpy as jnp
from jax import lax
from jax.experimental import pallas as pl
from jax.experimental.pallas import tpu as pltpu
from jax.experimental.pallas import tpu_sc as plsc

N = 10000
E = 320000
NC, NS = 2, 16
NW = NC * NS
CHUNK = 80
ROW_T = 640
ROW_LAST = N - (NS - 1) * ROW_T

_f32 = jnp.float32
_i32 = jnp.int32


def _mesh():
    return plsc.VectorSubcoreMesh(core_axis_name="c", subcore_axis_name="s")


_SC_PARAMS = pltpu.CompilerParams(needs_layout_passes=False)


def _zero_rows_2d(zb, d):
    z16 = jnp.zeros((16,), _f32)

    def body(i, _):
        for j in range(d // 16):
            zb[i, pl.ds(j * 16, 16)] = z16
        return 0

    lax.fori_loop(0, 80, body, 0)


def _zero_1d(ref, n):
    z16 = jnp.zeros((16,), _f32)

    def body(i, _):
        ref[pl.ds(i * 16, 16)] = z16
        return 0

    lax.fori_loop(0, n // 16, body, 0)


def _rsqrt16(d):
    i = lax.bitcast_convert_type(d, _i32)
    i = 0x5F3759DF - lax.shift_right_logical(i, 1)
    y = lax.bitcast_convert_type(i, _f32)
    for _ in range(3):
        y = y * (1.5 - 0.5 * d * y * y)
    return y


def _degree_call(src, dst):
    EPT = E // NW
    NCH = EPT // CHUNK

    @functools.partial(
        pl.kernel,
        mesh=_mesh(),
        compiler_params=_SC_PARAMS,
        out_type=[jax.ShapeDtypeStruct((N,), _f32)] * 4,
        scratch_types=[
            pltpu.VMEM((CHUNK,), _i32),
            pltpu.VMEM((CHUNK,), _f32),
            pltpu.VMEM((ROW_T,), _f32),
            pltpu.VMEM_SHARED((N,), _f32),
            pltpu.VMEM_SHARED((N,), _f32),
        ],
    )
    def deg_k(src_hbm, dst_hbm, do0, di0, do1, di1, idx_v, ones_v, zb,
              acc_o, acc_i):
        cc = lax.axis_index("c")
        ss = lax.axis_index("s")
        one16 = jnp.ones((16,), _f32)
        for g in range(CHUNK // 16):
            ones_v[pl.ds(g * 16, 16)] = one16
        _zero_1d(zb, ROW_T)
        r0 = ss * ROW_T

        @pl.when(ss < NS - 1)
        def _():
            pltpu.sync_copy(zb, acc_o.at[pl.ds(r0, ROW_T)])
            pltpu.sync_copy(zb, acc_i.at[pl.ds(r0, ROW_T)])

        @pl.when(ss == NS - 1)
        def _():
            pltpu.sync_copy(zb.at[pl.ds(0, ROW_LAST)],
                            acc_o.at[pl.ds(r0, ROW_LAST)])
            pltpu.sync_copy(zb.at[pl.ds(0, ROW_LAST)],
                            acc_i.at[pl.ds(r0, ROW_LAST)])

        plsc.subcore_barrier()

        e0 = (cc * NS + ss) * EPT

        def chunk(k, _):
            base = e0 + k * CHUNK
            pltpu.sync_copy(src_hbm.at[pl.ds(base, CHUNK)], idx_v)
            pltpu.sync_copy(ones_v, acc_o.at[idx_v], add=True)
            pltpu.sync_copy(dst_hbm.at[pl.ds(base, CHUNK)], idx_v)
            pltpu.sync_copy(ones_v, acc_i.at[idx_v], add=True)
            return 0

        lax.fori_loop(0, NCH, chunk, 0)
        plsc.subcore_barrier()

        def wout(o_ref, i_ref, off, size):
            seg = pl.ds(off, size)
            bseg = pl.ds(0, size)
            pltpu.sync_copy(acc_o.at[seg], zb.at[bseg])
            pltpu.sync_copy(zb.at[bseg], o_ref.at[seg])
            pltpu.sync_copy(acc_i.at[seg], zb.at[bseg])
            pltpu.sync_copy(zb.at[bseg], i_ref.at[seg])

        @pl.when(jnp.logical_and(cc == 0, ss < NS - 1))
        def _():
            wout(do0, di0, r0, ROW_T)

        @pl.when(jnp.logical_and(cc == 0, ss == NS - 1))
        def _():
            wout(do0, di0, r0, ROW_LAST)

        @pl.when(jnp.logical_and(cc == 1, ss < NS - 1))
        def _():
            wout(do1, di1, r0, ROW_T)

        @pl.when(jnp.logical_and(cc == 1, ss == NS - 1))
        def _():
            wout(do1, di1, r0, ROW_LAST)

    return deg_k(src, dst)


def _coef_call(do0, di0, do1, di1, src, dst, ew):
    EPT = E // NW
    NCH = EPT // CHUNK

    @functools.partial(
        pl.kernel,
        mesh=_mesh(),
        compiler_params=_SC_PARAMS,
        out_type=jax.ShapeDtypeStruct((E,), _f32),
        scratch_types=[
            pltpu.VMEM((N,), _f32),
            pltpu.VMEM((N,), _f32),
            pltpu.VMEM((N,), _f32),
            pltpu.VMEM((N,), _f32),
            pltpu.VMEM((CHUNK,), _i32),
            pltpu.VMEM((CHUNK,), _i32),
            pltpu.VMEM((CHUNK,), _f32),
            pltpu.VMEM((CHUNK,), _f32),
        ],
    )
    def coef_k(do0_h, di0_h, do1_h, di1_h, src_h, dst_h, ew_h, c_h,
               pa, pb, no_v, ni_v, srcv, dstv, ewv, cv):
        cc = lax.axis_index("c")
        ss = lax.axis_index("s")
        pltpu.sync_copy(do0_h, pa)
        pltpu.sync_copy(do1_h, pb)

        def nbody(i, _):
            sl = pl.ds(i * 16, 16)
            no_v[sl] = _rsqrt16(jnp.maximum(pa[sl] + pb[sl], 1.0))
            return 0

        lax.fori_loop(0, N // 16, nbody, 0)
        pltpu.sync_copy(di0_h, pa)
        pltpu.sync_copy(di1_h, pb)

        def nbody2(i, _):
            sl = pl.ds(i * 16, 16)
            ni_v[sl] = _rsqrt16(jnp.maximum(pa[sl] + pb[sl], 1.0))
            return 0

        lax.fori_loop(0, N // 16, nbody2, 0)

        e0 = (cc * NS + ss) * EPT

        def chunk(k, _):
            base = e0 + k * CHUNK
            pltpu.sync_copy(src_h.at[pl.ds(base, CHUNK)], srcv)
            pltpu.sync_copy(dst_h.at[pl.ds(base, CHUNK)], dstv)
            pltpu.sync_copy(ew_h.at[pl.ds(base, CHUNK)], ewv)
            for g in range(CHUNK // 16):
                sl = pl.ds(g * 16, 16)
                no16 = plsc.load_gather(no_v, [srcv[sl]])
                ni16 = plsc.load_gather(ni_v, [dstv[sl]])
                cv[sl] = ewv[sl] * no16 * ni16
            pltpu.sync_copy(cv, c_h.at[pl.ds(base, CHUNK)])
            return 0

        lax.fori_loop(0, NCH, chunk, 0)

    return coef_k(do0, di0, do1, di1, src, dst, ew)


def _msgpass_call(h_args, src, dst, c, d, dacc, col_split):
    EPT = E // NS if col_split else E // NW
    NCH = EPT // CHUNK
    NG = CHUNK // 16
    NM = NCH // 2
    EPI = NCH % 2 == 1
    PRE_C = EPT <= 10000
    SEP_S = dacc != d

    scratch = [
        pltpu.VMEM((EPT,), _i32),
        pltpu.VMEM((CHUNK,), _i32),
        pltpu.VMEM((CHUNK,), _i32),
        pltpu.VMEM((CHUNK,), _i32),
        pltpu.VMEM((CHUNK,), _i32),
        pltpu.VMEM((CHUNK, d), _f32),
        pltpu.VMEM((CHUNK, d), _f32),
        pltpu.VMEM_SHARED((N, dacc), _f32),
        pltpu.SemaphoreType.DMA,
        pltpu.SemaphoreType.DMA,
        pltpu.SemaphoreType.DMA,
        pltpu.SemaphoreType.DMA,
    ]
    if PRE_C:
        scratch.append(pltpu.VMEM((EPT,), _f32))
    else:
        scratch += [pltpu.VMEM((CHUNK,), _f32),
                    pltpu.VMEM((CHUNK,), _f32),
                    pltpu.SemaphoreType.DMA,
                    pltpu.SemaphoreType.DMA]
    if SEP_S:
        scratch.append(pltpu.VMEM((CHUNK, dacc), _f32))

    @functools.partial(
        pl.kernel,
        mesh=_mesh(),
        compiler_params=_SC_PARAMS,
        out_type=[jax.ShapeDtypeStruct((N, dacc), _f32)] * 2,
        scratch_types=scratch,
    )
    def mp_k(*refs):
        if col_split:
            ha, hb, src_h, dst_h, c_h = refs[:5]
            outs = refs[5:7]
            rest = list(refs[7:])
        else:
            h_h, src_h, dst_h, c_h = refs[:4]
            outs = refs[4:6]
            rest = list(refs[6:])
        (srcall, sA, sB, dA, dB, rowsA, rowsB, acc,
         semA, semB, semDA, semDB) = rest[:12]
        rest = rest[12:]
        if PRE_C:
            call_ = rest.pop(0)
            cvA = cvB = semCA = semCB = None
        else:
            cvA, cvB, semCA, semCB = rest[:4]
            rest = rest[4:]
        rows_s = rest.pop(0) if SEP_S else None
        zb = rows_s if SEP_S else rowsB
        cc = lax.axis_index("c")
        ss = lax.axis_index("s")

        _zero_rows_2d(zb, dacc)
        r0 = ss * ROW_T

        def zero_acc(nseg):
            def zbody(k, _):
                pltpu.sync_copy(zb, acc.at[pl.ds(r0 + k * 80, 80)])
                return 0
            lax.fori_loop(0, nseg, zbody, 0)

        @pl.when(ss < NS - 1)
        def _():
            zero_acc(ROW_T // 80)

        @pl.when(ss == NS - 1)
        def _():
            zero_acc(ROW_LAST // 80)

        plsc.subcore_barrier()

        e0 = (ss if col_split else cc * NS + ss) * EPT

        def cslice(dref, sref, k):
            for t in range(NG):
                dref[pl.ds(t * 16, 16)] = sref[pl.ds(k * CHUNK + t * 16, 16)]

        def edge_loop(h_ref):
            pltpu.sync_copy(src_h.at[pl.ds(e0, EPT)], srcall)
            if PRE_C:
                pltpu.sync_copy(c_h.at[pl.ds(e0, EPT)], call_)

            def start(k, sX, rowsX, semX, dX, semDX, cvX, semCX):
                cslice(sX, srcall, k)
                pltpu.async_copy(h_ref.at[sX], rowsX, semX)
                seg = pl.ds(e0 + k * CHUNK, CHUNK)
                pltpu.async_copy(dst_h.at[seg], dX, semDX)
                if not PRE_C:
                    pltpu.async_copy(c_h.at[seg], cvX, semCX)

            def finish(k, sX, rowsX, semX, dX, semDX, cvX, semCX):
                pltpu.make_async_copy(h_ref.at[sX], rowsX, semX).wait()
                if not PRE_C:
                    pltpu.make_async_copy(
                        c_h.at[pl.ds(e0 + k * CHUNK, CHUNK)], cvX,
                        semCX).wait()
                base = k * CHUNK
                out_rows = rows_s if SEP_S else rowsX

                def gbody(g, _):
                    for r in range(16):
                        i = g * 16 + r
                        if PRE_C:
                            ci = plsc.load_gather(
                                call_, [jnp.broadcast_to(base + i, (16,))])
                        else:
                            ci = plsc.load_gather(
                                cvX, [jnp.broadcast_to(i, (16,))])
                        for j in range(dacc // 16):
                            sl = pl.ds(j * 16, 16)
                            out_rows[i, sl] = rowsX[i, sl] * ci
                    return 0

                lax.fori_loop(0, NG, gbody, 0)
                pltpu.make_async_copy(
                    dst_h.at[pl.ds(e0 + k * CHUNK, CHUNK)], dX,
                    semDX).wait()
                pltpu.sync_copy(out_rows, acc.at[dX], add=True)

            A = (sA, rowsA, semA, dA, semDA, cvA, semCA)
            B = (sB, rowsB, semB, dB, semDB, cvB, semCB)
            start(0, *A)

            def pair(m, _):
                start(2 * m + 1, *B)
                finish(2 * m, *A)
                if EPI:
                    start(2 * m + 2, *A)
                else:
                    @pl.when(m < NM - 1)
                    def _():
                        start(2 * m + 2, *A)
                finish(2 * m + 1, *B)
                return 0

            lax.fori_loop(0, NM, pair, 0)
            if EPI:
                finish(NCH - 1, *A)

        if col_split:
            @pl.when(cc == 0)
            def _():
                edge_loop(ha)

            @pl.when(cc == 1)
            def _():
                edge_loop(hb)
        else:
            edge_loop(h_h)

        plsc.subcore_barrier()

        def wout(o_ref, off, size):
            def wbody(k, _):
                seg = pl.ds(off + k * 80, 80)
                pltpu.sync_copy(acc.at[seg], zb)
                pltpu.sync_copy(zb, o_ref.at[seg])
                return 0
            lax.fori_loop(0, size // 80, wbody, 0)

        @pl.when(jnp.logical_and(cc == 0, ss < NS - 1))
        def _():
            wout(outs[0], r0, ROW_T)

        @pl.when(jnp.logical_and(cc == 0, ss == NS - 1))
        def _():
            wout(outs[0], r0, ROW_LAST)

        @pl.when(jnp.logical_and(cc == 1, ss < NS - 1))
        def _():
            wout(outs[1], r0, ROW_T)

        @pl.when(jnp.logical_and(cc == 1, ss == NS - 1))
        def _():
            wout(outs[1], r0, ROW_LAST)

    return mp_k(*h_args, src, dst, c)


_BLK = 400


def _tc_l1(p0, p1, W1, b1):
    def body(p0_r, p1_r, w_r, b_r, oa_r, ob_r):
        a = p0_r[...] + p1_r[...]
        h = jnp.dot(a, w_r[...], preferred_element_type=_f32) + b_r[...]
        h = jnp.maximum(h, 0.0)
        oa_r[...] = h[:, :128]
        ob_r[...] = h[:, 128:]

    return pl.pallas_call(
        body,
        grid=(N // _BLK,),
        in_specs=[
            pl.BlockSpec((_BLK, 128), lambda r: (r, 0)),
            pl.BlockSpec((_BLK, 128), lambda r: (r, 0)),
            pl.BlockSpec((128, 256), lambda r: (0, 0)),
            pl.BlockSpec((1, 256), lambda r: (0, 0)),
        ],
        out_specs=[pl.BlockSpec((_BLK, 128), lambda r: (r, 0))] * 2,
        out_shape=[jax.ShapeDtypeStruct((N, 128), _f32)] * 2,
    )(p0, p1, W1, b1)


def _tc_l2l3(aa, ab, W2, b2, W3):
    def body(aa_r, ab_r, w2_r, b2_r, w3_r, o_r):
        h = (jnp.dot(aa_r[...], w2_r[:128, :], preferred_element_type=_f32)
             + jnp.dot(ab_r[...], w2_r[128:, :], preferred_element_type=_f32)
             + b2_r[...])
        h = jnp.maximum(h, 0.0)
        t = jnp.dot(h, w3_r[...], preferred_element_type=_f32)
        o_r[...] = jnp.concatenate([t, jnp.zeros_like(t)], axis=1)

    return pl.pallas_call(
        body,
        grid=(N // _BLK,),
        in_specs=[
            pl.BlockSpec((_BLK, 128), lambda r: (r, 0)),
            pl.BlockSpec((_BLK, 128), lambda r: (r, 0)),
            pl.BlockSpec((256, 256), lambda r: (0, 0)),
            pl.BlockSpec((1, 256), lambda r: (0, 0)),
            pl.BlockSpec((256, 64), lambda r: (0, 0)),
        ],
        out_specs=pl.BlockSpec((_BLK, 128), lambda r: (r, 0)),
        out_shape=jax.ShapeDtypeStruct((N, 128), _f32),
    )(aa, ab, W2, b2, W3)


def _tc_out(p0, p1, b3):
    def body(p0_r, p1_r, b_r, o_r):
        z = p0_r[...] + p1_r[...] + b_r[...]
        m = jnp.max(z, axis=-1, keepdims=True)
        e = jnp.exp(z - m)
        o_r[...] = e / jnp.sum(e, axis=-1, keepdims=True)

    return pl.pallas_call(
        body,
        grid=(N // _BLK,),
        in_specs=[
            pl.BlockSpec((_BLK, 64), lambda r: (r, 0)),
            pl.BlockSpec((_BLK, 64), lambda r: (r, 0)),
            pl.BlockSpec((1, 64), lambda r: (0, 0)),
        ],
        out_specs=pl.BlockSpec((_BLK, 64), lambda r: (r, 0)),
        out_shape=jax.ShapeDtypeStruct((N, 64), _f32),
    )(p0, p1, b3)


def kernel(features, edge_index, edge_weight, W1, b1, W2, b2, W3, b3):
    src = edge_index[0]
    dst = edge_index[1]
    do0, di0, do1, di1 = _degree_call(src, dst)
    c = _coef_call(do0, di0, do1, di1, src, dst, edge_weight)
    p1a, p1b = _msgpass_call((features,), src, dst, c, 128, 128,
                             col_split=False)
    h1a, h1b = _tc_l1(p1a, p1b, W1, b1.reshape(1, -1))
    a2a, a2b = _msgpass_call((h1a, h1b), src, dst, c, 128, 128,
                             col_split=True)
    t = _tc_l2l3(a2a, a2b, W2, b2.reshape(1, -1), W3)
    p3a, p3b = _msgpass_call((t,), src, dst, c, 128, 128, col_split=False)
    return _tc_out(p3a[:, :64], p3b[:, :64], b3.reshape(1, -1))


# speedup vs baseline: 8.4527x; 1.3637x over previous
import functools

import jax
import jax.numpy as jnp
from jax import lax
from jax.experimental import pallas as pl
from jax.experimental.pallas import tpu as pltpu
from jax.experimental.pallas import tpu_sc as plsc

N = 10000
E = 320000
NC, NS = 2, 16
NW = NC * NS
CHUNK = 80
ROW_T = 640
ROW_LAST = N - (NS - 1) * ROW_T

_f32 = jnp.float32
_i32 = jnp.int32


def _mesh():
    return plsc.VectorSubcoreMesh(core_axis_name="c", subcore_axis_name="s")


_SC_PARAMS = pltpu.CompilerParams(needs_layout_passes=False)


def _zero_rows_2d(zb, d):
    z16 = jnp.zeros((16,), _f32)

    def body(i, _):
        for j in range(d // 16):
            zb[i, pl.ds(j * 16, 16)] = z16
        return 0

    lax.fori_loop(0, 80, body, 0)


def _zero_1d(ref, n):
    z16 = jnp.zeros((16,), _f32)

    def body(i, _):
        ref[pl.ds(i * 16, 16)] = z16
        return 0

    lax.fori_loop(0, n // 16, body, 0)


def _rsqrt16(d):
    i = lax.bitcast_convert_type(d, _i32)
    i = 0x5F3759DF - lax.shift_right_logical(i, 1)
    y = lax.bitcast_convert_type(i, _f32)
    for _ in range(3):
        y = y * (1.5 - 0.5 * d * y * y)
    return y


def _degree_call(src, dst):
    EPT = E // NS

    @functools.partial(
        pl.kernel,
        mesh=_mesh(),
        compiler_params=_SC_PARAMS,
        out_type=[jax.ShapeDtypeStruct((N,), _f32)] * 2,
        scratch_types=[
            pltpu.VMEM((EPT,), _i32),
            pltpu.VMEM((EPT,), _f32),
            pltpu.VMEM((ROW_T,), _f32),
            pltpu.VMEM_SHARED((N,), _f32),
        ],
    )
    def deg_k(src_hbm, dst_hbm, do_h, di_h, idxall, ones_v, zb, acc):
        cc = lax.axis_index("c")
        ss = lax.axis_index("s")
        one16 = jnp.ones((16,), _f32)

        def fill(i, _):
            ones_v[pl.ds(i * 16, 16)] = one16
            return 0

        lax.fori_loop(0, EPT // 16, fill, 0)
        _zero_1d(zb, ROW_T)
        r0 = ss * ROW_T

        @pl.when(ss < NS - 1)
        def _():
            pltpu.sync_copy(zb, acc.at[pl.ds(r0, ROW_T)])

        @pl.when(ss == NS - 1)
        def _():
            pltpu.sync_copy(zb.at[pl.ds(0, ROW_LAST)],
                            acc.at[pl.ds(r0, ROW_LAST)])

        e0 = ss * EPT

        @pl.when(cc == 0)
        def _():
            pltpu.sync_copy(src_hbm.at[pl.ds(e0, EPT)], idxall)

        @pl.when(cc == 1)
        def _():
            pltpu.sync_copy(dst_hbm.at[pl.ds(e0, EPT)], idxall)

        plsc.subcore_barrier()
        pltpu.sync_copy(ones_v, acc.at[idxall], add=True)
        plsc.subcore_barrier()

        def wout(o_ref, off, size):
            seg = pl.ds(off, size)
            bseg = pl.ds(0, size)
            pltpu.sync_copy(acc.at[seg], zb.at[bseg])
            pltpu.sync_copy(zb.at[bseg], o_ref.at[seg])

        @pl.when(jnp.logical_and(cc == 0, ss < NS - 1))
        def _():
            wout(do_h, r0, ROW_T)

        @pl.when(jnp.logical_and(cc == 0, ss == NS - 1))
        def _():
            wout(do_h, r0, ROW_LAST)

        @pl.when(jnp.logical_and(cc == 1, ss < NS - 1))
        def _():
            wout(di_h, r0, ROW_T)

        @pl.when(jnp.logical_and(cc == 1, ss == NS - 1))
        def _():
            wout(di_h, r0, ROW_LAST)

    return deg_k(src, dst)


def _coef_call(do, di, src, dst, ew):
    EPT = E // NW

    @functools.partial(
        pl.kernel,
        mesh=_mesh(),
        compiler_params=_SC_PARAMS,
        out_type=jax.ShapeDtypeStruct((E,), _f32),
        scratch_types=[
            pltpu.VMEM((ROW_T,), _f32),
            pltpu.VMEM((ROW_T,), _f32),
            pltpu.VMEM((N,), _f32),
            pltpu.VMEM((N,), _f32),
            pltpu.VMEM((EPT,), _i32),
            pltpu.VMEM((EPT,), _i32),
            pltpu.VMEM((EPT,), _f32),
            pltpu.VMEM((EPT,), _f32),
            pltpu.VMEM_SHARED((N,), _f32),
            pltpu.VMEM_SHARED((N,), _f32),
        ],
    )
    def coef_k(do_h, di_h, src_h, dst_h, ew_h, c_h,
               dsl, nsl, no_v, ni_v, srcall, dstall, ewall, cvall,
               no_s, ni_s):
        cc = lax.axis_index("c")
        ss = lax.axis_index("s")
        r0 = ss * ROW_T

        def build(deg_h, table_s, nrow):
            pltpu.sync_copy(deg_h.at[pl.ds(r0, nrow)], dsl.at[pl.ds(0, nrow)])

            def nbody(i, _):
                sl = pl.ds(i * 16, 16)
                nsl[sl] = _rsqrt16(jnp.maximum(dsl[sl], 1.0))
                return 0

            lax.fori_loop(0, nrow // 16, nbody, 0)
            pltpu.sync_copy(nsl.at[pl.ds(0, nrow)],
                            table_s.at[pl.ds(r0, nrow)])

        @pl.when(ss < NS - 1)
        def _():
            build(do_h, no_s, ROW_T)
            build(di_h, ni_s, ROW_T)

        @pl.when(ss == NS - 1)
        def _():
            build(do_h, no_s, ROW_LAST)
            build(di_h, ni_s, ROW_LAST)

        e0 = (cc * NS + ss) * EPT
        pltpu.sync_copy(src_h.at[pl.ds(e0, EPT)], srcall)
        pltpu.sync_copy(dst_h.at[pl.ds(e0, EPT)], dstall)
        pltpu.sync_copy(ew_h.at[pl.ds(e0, EPT)], ewall)
        plsc.subcore_barrier()
        pltpu.sync_copy(no_s, no_v)
        pltpu.sync_copy(ni_s, ni_v)

        def group(g, _):
            sl = pl.ds(g * 16, 16)
            no16 = plsc.load_gather(no_v, [srcall[sl]])
            ni16 = plsc.load_gather(ni_v, [dstall[sl]])
            cvall[sl] = ewall[sl] * no16 * ni16
            return 0

        lax.fori_loop(0, EPT // 16, group, 0)
        pltpu.sync_copy(cvall, c_h.at[pl.ds(e0, EPT)])

    return coef_k(do, di, src, dst, ew)


def _msgpass_call(h_args, src, dst, c, d, dacc, col_split):
    EPT = E // NS if col_split else E // NW
    NCH = EPT // CHUNK
    NG = CHUNK // 16
    NM = NCH // 2
    EPI = NCH % 2 == 1
    PRE_C = EPT <= 10000
    SEP_S = dacc != d

    scratch = [
        pltpu.VMEM((EPT,), _i32),
        pltpu.VMEM((CHUNK,), _i32),
        pltpu.VMEM((CHUNK,), _i32),
        pltpu.VMEM((CHUNK,), _i32),
        pltpu.VMEM((CHUNK,), _i32),
        pltpu.VMEM((CHUNK, d), _f32),
        pltpu.VMEM((CHUNK, d), _f32),
        pltpu.VMEM_SHARED((N, dacc), _f32),
        pltpu.SemaphoreType.DMA,
        pltpu.SemaphoreType.DMA,
        pltpu.SemaphoreType.DMA,
        pltpu.SemaphoreType.DMA,
    ]
    if PRE_C:
        scratch.append(pltpu.VMEM((EPT,), _f32))
    else:
        scratch += [pltpu.VMEM((CHUNK,), _f32),
                    pltpu.VMEM((CHUNK,), _f32),
                    pltpu.SemaphoreType.DMA,
                    pltpu.SemaphoreType.DMA]
    if SEP_S:
        scratch.append(pltpu.VMEM((CHUNK, dacc), _f32))

    @functools.partial(
        pl.kernel,
        mesh=_mesh(),
        compiler_params=_SC_PARAMS,
        out_type=[jax.ShapeDtypeStruct((N, dacc), _f32)] * 2,
        scratch_types=scratch,
    )
    def mp_k(*refs):
        if col_split:
            ha, hb, src_h, dst_h, c_h = refs[:5]
            outs = refs[5:7]
            rest = list(refs[7:])
        else:
            h_h, src_h, dst_h, c_h = refs[:4]
            outs = refs[4:6]
            rest = list(refs[6:])
        (srcall, sA, sB, dA, dB, rowsA, rowsB, acc,
         semA, semB, semDA, semDB) = rest[:12]
        rest = rest[12:]
        if PRE_C:
            call_ = rest.pop(0)
            cvA = cvB = semCA = semCB = None
        else:
            cvA, cvB, semCA, semCB = rest[:4]
            rest = rest[4:]
        rows_s = rest.pop(0) if SEP_S else None
        zb = rows_s if SEP_S else rowsB
        cc = lax.axis_index("c")
        ss = lax.axis_index("s")

        _zero_rows_2d(zb, dacc)
        r0 = ss * ROW_T

        def zero_acc(nseg):
            def zbody(k, _):
                pltpu.sync_copy(zb, acc.at[pl.ds(r0 + k * 80, 80)])
                return 0
            lax.fori_loop(0, nseg, zbody, 0)

        @pl.when(ss < NS - 1)
        def _():
            zero_acc(ROW_T // 80)

        @pl.when(ss == NS - 1)
        def _():
            zero_acc(ROW_LAST // 80)

        plsc.subcore_barrier()

        e0 = (ss if col_split else cc * NS + ss) * EPT

        def cslice(dref, sref, k):
            for t in range(NG):
                dref[pl.ds(t * 16, 16)] = sref[pl.ds(k * CHUNK + t * 16, 16)]

        def edge_loop(h_ref):
            pltpu.sync_copy(src_h.at[pl.ds(e0, EPT)], srcall)
            if PRE_C:
                pltpu.sync_copy(c_h.at[pl.ds(e0, EPT)], call_)

            def start(k, sX, rowsX, semX, dX, semDX, cvX, semCX):
                cslice(sX, srcall, k)
                pltpu.async_copy(h_ref.at[sX], rowsX, semX)
                seg = pl.ds(e0 + k * CHUNK, CHUNK)
                pltpu.async_copy(dst_h.at[seg], dX, semDX)
                if not PRE_C:
                    pltpu.async_copy(c_h.at[seg], cvX, semCX)

            def finish(k, sX, rowsX, semX, dX, semDX, cvX, semCX):
                pltpu.make_async_copy(h_ref.at[sX], rowsX, semX).wait()
                if not PRE_C:
                    pltpu.make_async_copy(
                        c_h.at[pl.ds(e0 + k * CHUNK, CHUNK)], cvX,
                        semCX).wait()
                base = k * CHUNK
                out_rows = rows_s if SEP_S else rowsX

                def gbody(g, _):
                    for r in range(16):
                        i = g * 16 + r
                        if PRE_C:
                            ci = plsc.load_gather(
                                call_, [jnp.broadcast_to(base + i, (16,))])
                        else:
                            ci = plsc.load_gather(
                                cvX, [jnp.broadcast_to(i, (16,))])
                        for j in range(dacc // 16):
                            sl = pl.ds(j * 16, 16)
                            out_rows[i, sl] = rowsX[i, sl] * ci
                    return 0

                lax.fori_loop(0, NG, gbody, 0)
                pltpu.make_async_copy(
                    dst_h.at[pl.ds(e0 + k * CHUNK, CHUNK)], dX,
                    semDX).wait()
                pltpu.sync_copy(out_rows, acc.at[dX], add=True)

            A = (sA, rowsA, semA, dA, semDA, cvA, semCA)
            B = (sB, rowsB, semB, dB, semDB, cvB, semCB)
            start(0, *A)

            def pair(m, _):
                start(2 * m + 1, *B)
                finish(2 * m, *A)
                if EPI:
                    start(2 * m + 2, *A)
                else:
                    @pl.when(m < NM - 1)
                    def _():
                        start(2 * m + 2, *A)
                finish(2 * m + 1, *B)
                return 0

            lax.fori_loop(0, NM, pair, 0)
            if EPI:
                finish(NCH - 1, *A)

        if col_split:
            @pl.when(cc == 0)
            def _():
                edge_loop(ha)

            @pl.when(cc == 1)
            def _():
                edge_loop(hb)
        else:
            edge_loop(h_h)

        plsc.subcore_barrier()

        def wout(o_ref, off, size):
            def wbody(k, _):
                seg = pl.ds(off + k * 80, 80)
                pltpu.sync_copy(acc.at[seg], zb)
                pltpu.sync_copy(zb, o_ref.at[seg])
                return 0
            lax.fori_loop(0, size // 80, wbody, 0)

        @pl.when(jnp.logical_and(cc == 0, ss < NS - 1))
        def _():
            wout(outs[0], r0, ROW_T)

        @pl.when(jnp.logical_and(cc == 0, ss == NS - 1))
        def _():
            wout(outs[0], r0, ROW_LAST)

        @pl.when(jnp.logical_and(cc == 1, ss < NS - 1))
        def _():
            wout(outs[1], r0, ROW_T)

        @pl.when(jnp.logical_and(cc == 1, ss == NS - 1))
        def _():
            wout(outs[1], r0, ROW_LAST)

    return mp_k(*h_args, src, dst, c)


_BLK = 400


def _tc_l1(p0, p1, W1, b1):
    def body(p0_r, p1_r, w_r, b_r, oa_r, ob_r):
        a = p0_r[...] + p1_r[...]
        h = jnp.dot(a, w_r[...], preferred_element_type=_f32) + b_r[...]
        h = jnp.maximum(h, 0.0)
        oa_r[...] = h[:, :128]
        ob_r[...] = h[:, 128:]

    return pl.pallas_call(
        body,
        grid=(N // _BLK,),
        in_specs=[
            pl.BlockSpec((_BLK, 128), lambda r: (r, 0)),
            pl.BlockSpec((_BLK, 128), lambda r: (r, 0)),
            pl.BlockSpec((128, 256), lambda r: (0, 0)),
            pl.BlockSpec((1, 256), lambda r: (0, 0)),
        ],
        out_specs=[pl.BlockSpec((_BLK, 128), lambda r: (r, 0))] * 2,
        out_shape=[jax.ShapeDtypeStruct((N, 128), _f32)] * 2,
    )(p0, p1, W1, b1)


def _tc_l2l3(aa, ab, W2, b2, W3):
    def body(aa_r, ab_r, w2_r, b2_r, w3_r, o_r):
        h = (jnp.dot(aa_r[...], w2_r[:128, :], preferred_element_type=_f32)
             + jnp.dot(ab_r[...], w2_r[128:, :], preferred_element_type=_f32)
             + b2_r[...])
        h = jnp.maximum(h, 0.0)
        t = jnp.dot(h, w3_r[...], preferred_element_type=_f32)
        o_r[...] = jnp.concatenate([t, jnp.zeros_like(t)], axis=1)

    return pl.pallas_call(
        body,
        grid=(N // _BLK,),
        in_specs=[
            pl.BlockSpec((_BLK, 128), lambda r: (r, 0)),
            pl.BlockSpec((_BLK, 128), lambda r: (r, 0)),
            pl.BlockSpec((256, 256), lambda r: (0, 0)),
            pl.BlockSpec((1, 256), lambda r: (0, 0)),
            pl.BlockSpec((256, 64), lambda r: (0, 0)),
        ],
        out_specs=pl.BlockSpec((_BLK, 128), lambda r: (r, 0)),
        out_shape=jax.ShapeDtypeStruct((N, 128), _f32),
    )(aa, ab, W2, b2, W3)


def _tc_out(p0, p1, b3):
    def body(p0_r, p1_r, b_r, o_r):
        z = p0_r[...] + p1_r[...] + b_r[...]
        m = jnp.max(z, axis=-1, keepdims=True)
        e = jnp.exp(z - m)
        o_r[...] = e / jnp.sum(e, axis=-1, keepdims=True)

    return pl.pallas_call(
        body,
        grid=(N // _BLK,),
        in_specs=[
            pl.BlockSpec((_BLK, 64), lambda r: (r, 0)),
            pl.BlockSpec((_BLK, 64), lambda r: (r, 0)),
            pl.BlockSpec((1, 64), lambda r: (0, 0)),
        ],
        out_specs=pl.BlockSpec((_BLK, 64), lambda r: (r, 0)),
        out_shape=jax.ShapeDtypeStruct((N, 64), _f32),
    )(p0, p1, b3)


def kernel(features, edge_index, edge_weight, W1, b1, W2, b2, W3, b3):
    src = edge_index[0]
    dst = edge_index[1]
    do, di = _degree_call(src, dst)
    c = _coef_call(do, di, src, dst, edge_weight)
    p1a, p1b = _msgpass_call((features,), src, dst, c, 128, 128,
                             col_split=False)
    h1a, h1b = _tc_l1(p1a, p1b, W1, b1.reshape(1, -1))
    a2a, a2b = _msgpass_call((h1a, h1b), src, dst, c, 128, 128,
                             col_split=True)
    t = _tc_l2l3(a2a, a2b, W2, b2.reshape(1, -1), W3)
    p3a, p3b = _msgpass_call((t,), src, dst, c, 128, 128, col_split=False)
    return _tc_out(p3a[:, :64], p3b[:, :64], b3.reshape(1, -1))


# speedup vs baseline: 9.9200x; 1.1736x over previous
import functools

import jax
import jax.numpy as jnp
from jax import lax
from jax.experimental import pallas as pl
from jax.experimental.pallas import tpu as pltpu
from jax.experimental.pallas import tpu_sc as plsc

N = 10000
E = 320000
NC, NS = 2, 16
NW = NC * NS
CHUNK = 80
ROW_T = 640
ROW_LAST = N - (NS - 1) * ROW_T

_f32 = jnp.float32
_i32 = jnp.int32


def _mesh():
    return plsc.VectorSubcoreMesh(core_axis_name="c", subcore_axis_name="s")


_SC_PARAMS = pltpu.CompilerParams(needs_layout_passes=False)


def _zero_rows_2d(zb, d):
    z16 = jnp.zeros((16,), _f32)

    def body(i, _):
        for j in range(d // 16):
            zb[i, pl.ds(j * 16, 16)] = z16
        return 0

    lax.fori_loop(0, 80, body, 0)


def _zero_1d(ref, n):
    z16 = jnp.zeros((16,), _f32)

    def body(i, _):
        ref[pl.ds(i * 16, 16)] = z16
        return 0

    lax.fori_loop(0, n // 16, body, 0)


def _rsqrt16(d):
    i = lax.bitcast_convert_type(d, _i32)
    i = 0x5F3759DF - lax.shift_right_logical(i, 1)
    y = lax.bitcast_convert_type(i, _f32)
    for _ in range(3):
        y = y * (1.5 - 0.5 * d * y * y)
    return y


def _degree_call(src, dst):
    EPT = E // NS

    @functools.partial(
        pl.kernel,
        mesh=_mesh(),
        compiler_params=_SC_PARAMS,
        out_type=[jax.ShapeDtypeStruct((N,), _f32)] * 2,
        scratch_types=[
            pltpu.VMEM((EPT,), _i32),
            pltpu.VMEM((EPT,), _f32),
            pltpu.VMEM((ROW_T,), _f32),
            pltpu.VMEM_SHARED((N,), _f32),
        ],
    )
    def deg_k(src_hbm, dst_hbm, do_h, di_h, idxall, ones_v, zb, acc):
        cc = lax.axis_index("c")
        ss = lax.axis_index("s")
        one16 = jnp.ones((16,), _f32)

        def fill(i, _):
            ones_v[pl.ds(i * 16, 16)] = one16
            return 0

        lax.fori_loop(0, EPT // 16, fill, 0)
        _zero_1d(zb, ROW_T)
        r0 = ss * ROW_T

        @pl.when(ss < NS - 1)
        def _():
            pltpu.sync_copy(zb, acc.at[pl.ds(r0, ROW_T)])

        @pl.when(ss == NS - 1)
        def _():
            pltpu.sync_copy(zb.at[pl.ds(0, ROW_LAST)],
                            acc.at[pl.ds(r0, ROW_LAST)])

        e0 = ss * EPT

        @pl.when(cc == 0)
        def _():
            pltpu.sync_copy(src_hbm.at[pl.ds(e0, EPT)], idxall)

        @pl.when(cc == 1)
        def _():
            pltpu.sync_copy(dst_hbm.at[pl.ds(e0, EPT)], idxall)

        plsc.subcore_barrier()
        pltpu.sync_copy(ones_v, acc.at[idxall], add=True)
        plsc.subcore_barrier()

        def wout(o_ref, off, size):
            seg = pl.ds(off, size)
            bseg = pl.ds(0, size)
            pltpu.sync_copy(acc.at[seg], zb.at[bseg])
            pltpu.sync_copy(zb.at[bseg], o_ref.at[seg])

        @pl.when(jnp.logical_and(cc == 0, ss < NS - 1))
        def _():
            wout(do_h, r0, ROW_T)

        @pl.when(jnp.logical_and(cc == 0, ss == NS - 1))
        def _():
            wout(do_h, r0, ROW_LAST)

        @pl.when(jnp.logical_and(cc == 1, ss < NS - 1))
        def _():
            wout(di_h, r0, ROW_T)

        @pl.when(jnp.logical_and(cc == 1, ss == NS - 1))
        def _():
            wout(di_h, r0, ROW_LAST)

    return deg_k(src, dst)


def _coef_call(do, di, src, dst, ew):
    EPT = E // NW

    @functools.partial(
        pl.kernel,
        mesh=_mesh(),
        compiler_params=_SC_PARAMS,
        out_type=jax.ShapeDtypeStruct((E,), _f32),
        scratch_types=[
            pltpu.VMEM((ROW_T,), _f32),
            pltpu.VMEM((ROW_T,), _f32),
            pltpu.VMEM((N,), _f32),
            pltpu.VMEM((N,), _f32),
            pltpu.VMEM((EPT,), _i32),
            pltpu.VMEM((EPT,), _i32),
            pltpu.VMEM((EPT,), _f32),
            pltpu.VMEM((EPT,), _f32),
            pltpu.VMEM_SHARED((N,), _f32),
            pltpu.VMEM_SHARED((N,), _f32),
        ],
    )
    def coef_k(do_h, di_h, src_h, dst_h, ew_h, c_h,
               dsl, nsl, no_v, ni_v, srcall, dstall, ewall, cvall,
               no_s, ni_s):
        cc = lax.axis_index("c")
        ss = lax.axis_index("s")
        r0 = ss * ROW_T

        def build(deg_h, table_s, nrow):
            pltpu.sync_copy(deg_h.at[pl.ds(r0, nrow)], dsl.at[pl.ds(0, nrow)])

            def nbody(i, _):
                sl = pl.ds(i * 16, 16)
                nsl[sl] = _rsqrt16(jnp.maximum(dsl[sl], 1.0))
                return 0

            lax.fori_loop(0, nrow // 16, nbody, 0)
            pltpu.sync_copy(nsl.at[pl.ds(0, nrow)],
                            table_s.at[pl.ds(r0, nrow)])

        @pl.when(ss < NS - 1)
        def _():
            build(do_h, no_s, ROW_T)
            build(di_h, ni_s, ROW_T)

        @pl.when(ss == NS - 1)
        def _():
            build(do_h, no_s, ROW_LAST)
            build(di_h, ni_s, ROW_LAST)

        e0 = (cc * NS + ss) * EPT
        pltpu.sync_copy(src_h.at[pl.ds(e0, EPT)], srcall)
        pltpu.sync_copy(dst_h.at[pl.ds(e0, EPT)], dstall)
        pltpu.sync_copy(ew_h.at[pl.ds(e0, EPT)], ewall)
        plsc.subcore_barrier()
        pltpu.sync_copy(no_s, no_v)
        pltpu.sync_copy(ni_s, ni_v)

        def group(g, _):
            sl = pl.ds(g * 16, 16)
            no16 = plsc.load_gather(no_v, [srcall[sl]])
            ni16 = plsc.load_gather(ni_v, [dstall[sl]])
            cvall[sl] = ewall[sl] * no16 * ni16
            return 0

        lax.fori_loop(0, EPT // 16, group, 0)
        pltpu.sync_copy(cvall, c_h.at[pl.ds(e0, EPT)])

    return coef_k(do, di, src, dst, ew)


def _msgpass_call(h_args, src, dst, c, d, dacc, col_split):
    EPT = E // NS if col_split else E // NW
    NCH = EPT // CHUNK
    NG = CHUNK // 16
    NGRP = (NCH + 2) // 3

    scratch = []
    for _b in range(3):
        scratch += [
            pltpu.VMEM((CHUNK,), _i32),
            pltpu.VMEM((CHUNK,), _i32),
            pltpu.VMEM((CHUNK,), _i32),
            pltpu.VMEM((CHUNK,), _f32),
            pltpu.VMEM((CHUNK, d), _f32),
            pltpu.SemaphoreType.DMA,
            pltpu.SemaphoreType.DMA,
            pltpu.SemaphoreType.DMA,
        ]
    scratch.append(pltpu.VMEM_SHARED((N, dacc), _f32))

    @functools.partial(
        pl.kernel,
        mesh=_mesh(),
        compiler_params=_SC_PARAMS,
        out_type=[jax.ShapeDtypeStruct((N, dacc), _f32)] * 2,
        scratch_types=scratch,
    )
    def mp_k(*refs):
        if col_split:
            ha, hb, src_h, dst_h, c_h = refs[:5]
            outs = refs[5:7]
            rest = list(refs[7:])
        else:
            h_h, src_h, dst_h, c_h = refs[:4]
            outs = refs[4:6]
            rest = list(refs[6:])
        BUF = [tuple(rest[8 * b:8 * b + 8]) for b in range(3)]
        acc = rest[24]
        zb = BUF[0][4]
        cc = lax.axis_index("c")
        ss = lax.axis_index("s")

        _zero_rows_2d(zb, dacc)
        r0 = ss * ROW_T

        def zero_acc(nseg):
            def zbody(k, _):
                pltpu.sync_copy(zb, acc.at[pl.ds(r0 + k * 80, 80)])
                return 0
            lax.fori_loop(0, nseg, zbody, 0)

        @pl.when(ss < NS - 1)
        def _():
            zero_acc(ROW_T // 80)

        @pl.when(ss == NS - 1)
        def _():
            zero_acc(ROW_LAST // 80)

        plsc.subcore_barrier()

        e0 = (ss if col_split else cc * NS + ss) * EPT

        def edge_loop(h_ref):
            def start_idx(k, buf):
                s, dl, dsc, cv, rows, semI, semG, semS = buf
                seg = pl.ds(e0 + k * CHUNK, CHUNK)
                pltpu.async_copy(src_h.at[seg], s, semI)
                pltpu.async_copy(dst_h.at[seg], dl, semI)
                pltpu.async_copy(c_h.at[seg], cv, semI)

            def wait_idx(k, buf):
                s, dl, dsc, cv, rows, semI, semG, semS = buf
                seg = pl.ds(e0 + k * CHUNK, CHUNK)
                pltpu.make_async_copy(src_h.at[seg], s, semI).wait()
                pltpu.make_async_copy(dst_h.at[seg], dl, semI).wait()
                pltpu.make_async_copy(c_h.at[seg], cv, semI).wait()

            def start_gather(k, buf, wait_pred):
                s, dl, dsc, cv, rows, semI, semG, semS = buf
                if wait_pred is True:
                    pltpu.make_async_copy(rows, acc.at[dsc], semS).wait()
                elif wait_pred is not None:
                    @pl.when(wait_pred)
                    def _():
                        pltpu.make_async_copy(rows, acc.at[dsc],
                                              semS).wait()
                wait_idx(k, buf)
                pltpu.async_copy(h_ref.at[s], rows, semG)

            def finish(k, buf):
                s, dl, dsc, cv, rows, semI, semG, semS = buf
                pltpu.make_async_copy(h_ref.at[s], rows, semG).wait()

                def gbody(g, _):
                    for r in range(16):
                        i = g * 16 + r
                        ci = plsc.load_gather(
                            cv, [jnp.broadcast_to(i, (16,))])
                        for j in range(dacc // 16):
                            sl = pl.ds(j * 16, 16)
                            rows[i, sl] = rows[i, sl] * ci
                    return 0

                lax.fori_loop(0, NG, gbody, 0)
                for t in range(NG):
                    tsl = pl.ds(t * 16, 16)
                    dsc[tsl] = dl[tsl]
                pltpu.async_copy(rows, acc.at[dsc], semS, add=True)

            start_idx(0, BUF[0])
            start_idx(1, BUF[1])
            start_idx(2, BUF[2])
            start_gather(0, BUF[0], None)
            start_gather(1, BUF[1], None)

            def group(m, _):
                for r in range(3):
                    buf = BUF[r]
                    k = 3 * m + r

                    @pl.when(k < NCH)
                    def _():
                        finish(k, buf)

                    @pl.when(k + 3 < NCH)
                    def _():
                        start_idx(k + 3, buf)

                    @pl.when(k + 2 < NCH)
                    def _():
                        start_gather(k + 2, BUF[(r + 2) % 3],
                                     True if r != 0 else m >= 1)
                return 0

            lax.fori_loop(0, NGRP, group, 0)
            for r in range(3):
                k = NCH - 3 + r
                buf = BUF[k % 3]
                s, dl, dsc, cv, rows, semI, semG, semS = buf
                pltpu.make_async_copy(rows, acc.at[dsc], semS).wait()

        if col_split:
            @pl.when(cc == 0)
            def _():
                edge_loop(ha)

            @pl.when(cc == 1)
            def _():
                edge_loop(hb)
        else:
            edge_loop(h_h)

        plsc.subcore_barrier()

        def wout(o_ref, off, size):
            def wbody(k, _):
                seg = pl.ds(off + k * 80, 80)
                pltpu.sync_copy(acc.at[seg], zb)
                pltpu.sync_copy(zb, o_ref.at[seg])
                return 0
            lax.fori_loop(0, size // 80, wbody, 0)

        @pl.when(jnp.logical_and(cc == 0, ss < NS - 1))
        def _():
            wout(outs[0], r0, ROW_T)

        @pl.when(jnp.logical_and(cc == 0, ss == NS - 1))
        def _():
            wout(outs[0], r0, ROW_LAST)

        @pl.when(jnp.logical_and(cc == 1, ss < NS - 1))
        def _():
            wout(outs[1], r0, ROW_T)

        @pl.when(jnp.logical_and(cc == 1, ss == NS - 1))
        def _():
            wout(outs[1], r0, ROW_LAST)

    return mp_k(*h_args, src, dst, c)


_BLK = 400


def _tc_l1(p0, p1, W1, b1):
    def body(p0_r, p1_r, w_r, b_r, oa_r, ob_r):
        a = p0_r[...] + p1_r[...]
        h = jnp.dot(a, w_r[...], preferred_element_type=_f32) + b_r[...]
        h = jnp.maximum(h, 0.0)
        oa_r[...] = h[:, :128]
        ob_r[...] = h[:, 128:]

    return pl.pallas_call(
        body,
        grid=(N // _BLK,),
        in_specs=[
            pl.BlockSpec((_BLK, 128), lambda r: (r, 0)),
            pl.BlockSpec((_BLK, 128), lambda r: (r, 0)),
            pl.BlockSpec((128, 256), lambda r: (0, 0)),
            pl.BlockSpec((1, 256), lambda r: (0, 0)),
        ],
        out_specs=[pl.BlockSpec((_BLK, 128), lambda r: (r, 0))] * 2,
        out_shape=[jax.ShapeDtypeStruct((N, 128), _f32)] * 2,
    )(p0, p1, W1, b1)


def _tc_l2l3(aa, ab, W2, b2, W3):
    def body(aa_r, ab_r, w2_r, b2_r, w3_r, o_r):
        h = (jnp.dot(aa_r[...], w2_r[:128, :], preferred_element_type=_f32)
             + jnp.dot(ab_r[...], w2_r[128:, :], preferred_element_type=_f32)
             + b2_r[...])
        h = jnp.maximum(h, 0.0)
        t = jnp.dot(h, w3_r[...], preferred_element_type=_f32)
        o_r[...] = jnp.concatenate([t, jnp.zeros_like(t)], axis=1)

    return pl.pallas_call(
        body,
        grid=(N // _BLK,),
        in_specs=[
            pl.BlockSpec((_BLK, 128), lambda r: (r, 0)),
            pl.BlockSpec((_BLK, 128), lambda r: (r, 0)),
            pl.BlockSpec((256, 256), lambda r: (0, 0)),
            pl.BlockSpec((1, 256), lambda r: (0, 0)),
            pl.BlockSpec((256, 64), lambda r: (0, 0)),
        ],
        out_specs=pl.BlockSpec((_BLK, 128), lambda r: (r, 0)),
        out_shape=jax.ShapeDtypeStruct((N, 128), _f32),
    )(aa, ab, W2, b2, W3)


def _tc_out(p0, p1, b3):
    def body(p0_r, p1_r, b_r, o_r):
        z = p0_r[...] + p1_r[...] + b_r[...]
        m = jnp.max(z, axis=-1, keepdims=True)
        e = jnp.exp(z - m)
        o_r[...] = e / jnp.sum(e, axis=-1, keepdims=True)

    return pl.pallas_call(
        body,
        grid=(N // _BLK,),
        in_specs=[
            pl.BlockSpec((_BLK, 64), lambda r: (r, 0)),
            pl.BlockSpec((_BLK, 64), lambda r: (r, 0)),
            pl.BlockSpec((1, 64), lambda r: (0, 0)),
        ],
        out_specs=pl.BlockSpec((_BLK, 64), lambda r: (r, 0)),
        out_shape=jax.ShapeDtypeStruct((N, 64), _f32),
    )(p0, p1, b3)


def kernel(features, edge_index, edge_weight, W1, b1, W2, b2, W3, b3):
    src = edge_index[0]
    dst = edge_index[1]
    do, di = _degree_call(src, dst)
    c = _coef_call(do, di, src, dst, edge_weight)
    p1a, p1b = _msgpass_call((features,), src, dst, c, 128, 128,
                             col_split=False)
    h1a, h1b = _tc_l1(p1a, p1b, W1, b1.reshape(1, -1))
    a2a, a2b = _msgpass_call((h1a, h1b), src, dst, c, 128, 128,
                             col_split=True)
    t = _tc_l2l3(a2a, a2b, W2, b2.reshape(1, -1), W3)
    p3a, p3b = _msgpass_call((t,), src, dst, c, 128, 128, col_split=False)
    return _tc_out(p3a[:, :64], p3b[:, :64], b3.reshape(1, -1))


# speedup vs baseline: 11.0746x; 1.1164x over previous
import functools

import jax
import jax.numpy as jnp
from jax import lax
from jax.experimental import pallas as pl
from jax.experimental.pallas import tpu as pltpu
from jax.experimental.pallas import tpu_sc as plsc

N = 10000
E = 320000
NC, NS = 2, 16
NW = NC * NS
CHUNK = 80
ROW_T = 640
ROW_LAST = N - (NS - 1) * ROW_T

_f32 = jnp.float32
_i32 = jnp.int32


def _mesh():
    return plsc.VectorSubcoreMesh(core_axis_name="c", subcore_axis_name="s")


_SC_PARAMS = pltpu.CompilerParams(needs_layout_passes=False)


def _zero_rows_2d(zb, d):
    z16 = jnp.zeros((16,), _f32)

    def body(i, _):
        for j in range(d // 16):
            zb[i, pl.ds(j * 16, 16)] = z16
        return 0

    lax.fori_loop(0, 80, body, 0)


def _zero_1d(ref, n):
    z16 = jnp.zeros((16,), _f32)

    def body(i, _):
        ref[pl.ds(i * 16, 16)] = z16
        return 0

    lax.fori_loop(0, n // 16, body, 0)


def _rsqrt16(d):
    i = lax.bitcast_convert_type(d, _i32)
    i = 0x5F3759DF - lax.shift_right_logical(i, 1)
    y = lax.bitcast_convert_type(i, _f32)
    for _ in range(3):
        y = y * (1.5 - 0.5 * d * y * y)
    return y


def _degree_call(src, dst):
    EPT = E // NS

    @functools.partial(
        pl.kernel,
        mesh=_mesh(),
        compiler_params=_SC_PARAMS,
        out_type=[jax.ShapeDtypeStruct((N,), _f32)] * 2,
        scratch_types=[
            pltpu.VMEM((EPT,), _i32),
            pltpu.VMEM((EPT,), _f32),
            pltpu.VMEM((ROW_T,), _f32),
            pltpu.VMEM_SHARED((N,), _f32),
        ],
    )
    def deg_k(src_hbm, dst_hbm, do_h, di_h, idxall, ones_v, zb, acc):
        cc = lax.axis_index("c")
        ss = lax.axis_index("s")
        one16 = jnp.ones((16,), _f32)

        def fill(i, _):
            ones_v[pl.ds(i * 16, 16)] = one16
            return 0

        lax.fori_loop(0, EPT // 16, fill, 0)
        _zero_1d(zb, ROW_T)
        r0 = ss * ROW_T

        @pl.when(ss < NS - 1)
        def _():
            pltpu.sync_copy(zb, acc.at[pl.ds(r0, ROW_T)])

        @pl.when(ss == NS - 1)
        def _():
            pltpu.sync_copy(zb.at[pl.ds(0, ROW_LAST)],
                            acc.at[pl.ds(r0, ROW_LAST)])

        e0 = ss * EPT

        @pl.when(cc == 0)
        def _():
            pltpu.sync_copy(src_hbm.at[pl.ds(e0, EPT)], idxall)

        @pl.when(cc == 1)
        def _():
            pltpu.sync_copy(dst_hbm.at[pl.ds(e0, EPT)], idxall)

        plsc.subcore_barrier()
        pltpu.sync_copy(ones_v, acc.at[idxall], add=True)
        plsc.subcore_barrier()

        def wout(o_ref, off, size):
            seg = pl.ds(off, size)
            bseg = pl.ds(0, size)
            pltpu.sync_copy(acc.at[seg], zb.at[bseg])
            pltpu.sync_copy(zb.at[bseg], o_ref.at[seg])

        @pl.when(jnp.logical_and(cc == 0, ss < NS - 1))
        def _():
            wout(do_h, r0, ROW_T)

        @pl.when(jnp.logical_and(cc == 0, ss == NS - 1))
        def _():
            wout(do_h, r0, ROW_LAST)

        @pl.when(jnp.logical_and(cc == 1, ss < NS - 1))
        def _():
            wout(di_h, r0, ROW_T)

        @pl.when(jnp.logical_and(cc == 1, ss == NS - 1))
        def _():
            wout(di_h, r0, ROW_LAST)

    return deg_k(src, dst)


def _coef_call(do, di, src, dst, ew):
    EPT = E // NW

    @functools.partial(
        pl.kernel,
        mesh=_mesh(),
        compiler_params=_SC_PARAMS,
        out_type=jax.ShapeDtypeStruct((E,), _f32),
        scratch_types=[
            pltpu.VMEM((ROW_T,), _f32),
            pltpu.VMEM((ROW_T,), _f32),
            pltpu.VMEM((N,), _f32),
            pltpu.VMEM((N,), _f32),
            pltpu.VMEM((EPT,), _i32),
            pltpu.VMEM((EPT,), _i32),
            pltpu.VMEM((EPT,), _f32),
            pltpu.VMEM((EPT,), _f32),
            pltpu.VMEM_SHARED((N,), _f32),
            pltpu.VMEM_SHARED((N,), _f32),
        ],
    )
    def coef_k(do_h, di_h, src_h, dst_h, ew_h, c_h,
               dsl, nsl, no_v, ni_v, srcall, dstall, ewall, cvall,
               no_s, ni_s):
        cc = lax.axis_index("c")
        ss = lax.axis_index("s")
        r0 = ss * ROW_T

        def build(deg_h, table_s, nrow):
            pltpu.sync_copy(deg_h.at[pl.ds(r0, nrow)], dsl.at[pl.ds(0, nrow)])

            def nbody(i, _):
                sl = pl.ds(i * 16, 16)
                nsl[sl] = _rsqrt16(jnp.maximum(dsl[sl], 1.0))
                return 0

            lax.fori_loop(0, nrow // 16, nbody, 0)
            pltpu.sync_copy(nsl.at[pl.ds(0, nrow)],
                            table_s.at[pl.ds(r0, nrow)])

        @pl.when(ss < NS - 1)
        def _():
            build(do_h, no_s, ROW_T)
            build(di_h, ni_s, ROW_T)

        @pl.when(ss == NS - 1)
        def _():
            build(do_h, no_s, ROW_LAST)
            build(di_h, ni_s, ROW_LAST)

        e0 = (cc * NS + ss) * EPT
        pltpu.sync_copy(src_h.at[pl.ds(e0, EPT)], srcall)
        pltpu.sync_copy(dst_h.at[pl.ds(e0, EPT)], dstall)
        pltpu.sync_copy(ew_h.at[pl.ds(e0, EPT)], ewall)
        plsc.subcore_barrier()
        pltpu.sync_copy(no_s, no_v)
        pltpu.sync_copy(ni_s, ni_v)

        def group(g, _):
            sl = pl.ds(g * 16, 16)
            no16 = plsc.load_gather(no_v, [srcall[sl]])
            ni16 = plsc.load_gather(ni_v, [dstall[sl]])
            cvall[sl] = ewall[sl] * no16 * ni16
            return 0

        lax.fori_loop(0, EPT // 16, group, 0)
        pltpu.sync_copy(cvall, c_h.at[pl.ds(e0, EPT)])

    return coef_k(do, di, src, dst, ew)


def _msgpass_call(h_args, src, dst, c, d, dacc, col_split):
    EPT = E // NS if col_split else E // NW
    NCH = EPT // CHUNK
    NG = CHUNK // 16
    NGRP = (NCH + 3) // 4

    scratch = []
    for _b in range(4):
        scratch += [
            pltpu.VMEM((CHUNK,), _i32),
            pltpu.VMEM((CHUNK,), _i32),
            pltpu.VMEM((CHUNK,), _i32),
            pltpu.VMEM((CHUNK,), _f32),
            pltpu.VMEM((CHUNK, d), _f32),
            pltpu.SemaphoreType.DMA,
            pltpu.SemaphoreType.DMA,
            pltpu.SemaphoreType.DMA,
        ]
    scratch.append(pltpu.VMEM_SHARED((N, dacc), _f32))

    @functools.partial(
        pl.kernel,
        mesh=_mesh(),
        compiler_params=_SC_PARAMS,
        out_type=[jax.ShapeDtypeStruct((N, dacc), _f32)] * 2,
        scratch_types=scratch,
    )
    def mp_k(*refs):
        if col_split:
            ha, hb, src_h, dst_h, c_h = refs[:5]
            outs = refs[5:7]
            rest = list(refs[7:])
        else:
            h_h, src_h, dst_h, c_h = refs[:4]
            outs = refs[4:6]
            rest = list(refs[6:])
        BUF = [tuple(rest[8 * b:8 * b + 8]) for b in range(4)]
        acc = rest[32]
        zb = BUF[0][4]
        cc = lax.axis_index("c")
        ss = lax.axis_index("s")

        _zero_rows_2d(zb, dacc)
        r0 = ss * ROW_T

        def zero_acc(nseg):
            def zbody(k, _):
                pltpu.sync_copy(zb, acc.at[pl.ds(r0 + k * 80, 80)])
                return 0
            lax.fori_loop(0, nseg, zbody, 0)

        @pl.when(ss < NS - 1)
        def _():
            zero_acc(ROW_T // 80)

        @pl.when(ss == NS - 1)
        def _():
            zero_acc(ROW_LAST // 80)

        plsc.subcore_barrier()

        e0 = (ss if col_split else cc * NS + ss) * EPT

        def edge_loop(h_ref):
            def start_idx(k, buf):
                s, dl, dsc, cv, rows, semI, semG, semS = buf
                seg = pl.ds(e0 + k * CHUNK, CHUNK)
                pltpu.async_copy(src_h.at[seg], s, semI)
                pltpu.async_copy(dst_h.at[seg], dl, semI)
                pltpu.async_copy(c_h.at[seg], cv, semI)

            def wait_idx(k, buf):
                s, dl, dsc, cv, rows, semI, semG, semS = buf
                seg = pl.ds(e0 + k * CHUNK, CHUNK)
                pltpu.make_async_copy(src_h.at[seg], s, semI).wait()
                pltpu.make_async_copy(dst_h.at[seg], dl, semI).wait()
                pltpu.make_async_copy(c_h.at[seg], cv, semI).wait()

            def start_gather(k, buf, wait_pred):
                s, dl, dsc, cv, rows, semI, semG, semS = buf
                if wait_pred is True:
                    pltpu.make_async_copy(rows, acc.at[dsc], semS).wait()
                elif wait_pred is not None:
                    @pl.when(wait_pred)
                    def _():
                        pltpu.make_async_copy(rows, acc.at[dsc],
                                              semS).wait()
                wait_idx(k, buf)
                pltpu.async_copy(h_ref.at[s], rows, semG)

            def finish(k, buf):
                s, dl, dsc, cv, rows, semI, semG, semS = buf
                pltpu.make_async_copy(h_ref.at[s], rows, semG).wait()

                def gbody(g, _):
                    for r in range(16):
                        i = g * 16 + r
                        ci = plsc.load_gather(
                            cv, [jnp.broadcast_to(i, (16,))])
                        for j in range(dacc // 16):
                            sl = pl.ds(j * 16, 16)
                            rows[i, sl] = rows[i, sl] * ci
                    return 0

                lax.fori_loop(0, NG, gbody, 0)
                for t in range(NG):
                    tsl = pl.ds(t * 16, 16)
                    dsc[tsl] = dl[tsl]
                pltpu.async_copy(rows, acc.at[dsc], semS, add=True)

            start_idx(0, BUF[0])
            start_idx(1, BUF[1])
            start_idx(2, BUF[2])
            start_idx(3, BUF[3])
            start_gather(0, BUF[0], None)
            start_gather(1, BUF[1], None)
            start_gather(2, BUF[2], None)

            def group(m, _):
                for r in range(4):
                    buf = BUF[r]
                    k = 4 * m + r

                    @pl.when(k < NCH)
                    def _():
                        finish(k, buf)

                    @pl.when(k + 4 < NCH)
                    def _():
                        start_idx(k + 4, buf)

                    @pl.when(k + 3 < NCH)
                    def _():
                        start_gather(k + 3, BUF[(r + 3) % 4],
                                     True if r != 0 else m >= 1)
                return 0

            lax.fori_loop(0, NGRP, group, 0)
            for r in range(4):
                k = NCH - 4 + r
                buf = BUF[k % 4]
                s, dl, dsc, cv, rows, semI, semG, semS = buf
                pltpu.make_async_copy(rows, acc.at[dsc], semS).wait()

        if col_split:
            @pl.when(cc == 0)
            def _():
                edge_loop(ha)

            @pl.when(cc == 1)
            def _():
                edge_loop(hb)
        else:
            edge_loop(h_h)

        plsc.subcore_barrier()

        def wout(o_ref, off, size):
            def wbody(k, _):
                seg = pl.ds(off + k * 80, 80)
                pltpu.sync_copy(acc.at[seg], zb)
                pltpu.sync_copy(zb, o_ref.at[seg])
                return 0
            lax.fori_loop(0, size // 80, wbody, 0)

        @pl.when(jnp.logical_and(cc == 0, ss < NS - 1))
        def _():
            wout(outs[0], r0, ROW_T)

        @pl.when(jnp.logical_and(cc == 0, ss == NS - 1))
        def _():
            wout(outs[0], r0, ROW_LAST)

        @pl.when(jnp.logical_and(cc == 1, ss < NS - 1))
        def _():
            wout(outs[1], r0, ROW_T)

        @pl.when(jnp.logical_and(cc == 1, ss == NS - 1))
        def _():
            wout(outs[1], r0, ROW_LAST)

    return mp_k(*h_args, src, dst, c)


_BLK = 400


def _tc_l1(p0, p1, W1, b1):
    def body(p0_r, p1_r, w_r, b_r, oa_r, ob_r):
        a = p0_r[...] + p1_r[...]
        h = jnp.dot(a, w_r[...], preferred_element_type=_f32) + b_r[...]
        h = jnp.maximum(h, 0.0)
        oa_r[...] = h[:, :128]
        ob_r[...] = h[:, 128:]

    return pl.pallas_call(
        body,
        grid=(N // _BLK,),
        in_specs=[
            pl.BlockSpec((_BLK, 128), lambda r: (r, 0)),
            pl.BlockSpec((_BLK, 128), lambda r: (r, 0)),
            pl.BlockSpec((128, 256), lambda r: (0, 0)),
            pl.BlockSpec((1, 256), lambda r: (0, 0)),
        ],
        out_specs=[pl.BlockSpec((_BLK, 128), lambda r: (r, 0))] * 2,
        out_shape=[jax.ShapeDtypeStruct((N, 128), _f32)] * 2,
    )(p0, p1, W1, b1)


def _tc_l2l3(aa, ab, W2, b2, W3):
    def body(aa_r, ab_r, w2_r, b2_r, w3_r, o_r):
        h = (jnp.dot(aa_r[...], w2_r[:128, :], preferred_element_type=_f32)
             + jnp.dot(ab_r[...], w2_r[128:, :], preferred_element_type=_f32)
             + b2_r[...])
        h = jnp.maximum(h, 0.0)
        t = jnp.dot(h, w3_r[...], preferred_element_type=_f32)
        o_r[...] = jnp.concatenate([t, jnp.zeros_like(t)], axis=1)

    return pl.pallas_call(
        body,
        grid=(N // _BLK,),
        in_specs=[
            pl.BlockSpec((_BLK, 128), lambda r: (r, 0)),
            pl.BlockSpec((_BLK, 128), lambda r: (r, 0)),
            pl.BlockSpec((256, 256), lambda r: (0, 0)),
            pl.BlockSpec((1, 256), lambda r: (0, 0)),
            pl.BlockSpec((256, 64), lambda r: (0, 0)),
        ],
        out_specs=pl.BlockSpec((_BLK, 128), lambda r: (r, 0)),
        out_shape=jax.ShapeDtypeStruct((N, 128), _f32),
    )(aa, ab, W2, b2, W3)


def _tc_out(p0, p1, b3):
    def body(p0_r, p1_r, b_r, o_r):
        z = p0_r[...] + p1_r[...] + b_r[...]
        m = jnp.max(z, axis=-1, keepdims=True)
        e = jnp.exp(z - m)
        o_r[...] = e / jnp.sum(e, axis=-1, keepdims=True)

    return pl.pallas_call(
        body,
        grid=(N // _BLK,),
        in_specs=[
            pl.BlockSpec((_BLK, 64), lambda r: (r, 0)),
            pl.BlockSpec((_BLK, 64), lambda r: (r, 0)),
            pl.BlockSpec((1, 64), lambda r: (0, 0)),
        ],
        out_specs=pl.BlockSpec((_BLK, 64), lambda r: (r, 0)),
        out_shape=jax.ShapeDtypeStruct((N, 64), _f32),
    )(p0, p1, b3)


def kernel(features, edge_index, edge_weight, W1, b1, W2, b2, W3, b3):
    src = edge_index[0]
    dst = edge_index[1]
    do, di = _degree_call(src, dst)
    c = _coef_call(do, di, src, dst, edge_weight)
    p1a, p1b = _msgpass_call((features,), src, dst, c, 128, 128,
                             col_split=False)
    h1a, h1b = _tc_l1(p1a, p1b, W1, b1.reshape(1, -1))
    a2a, a2b = _msgpass_call((h1a, h1b), src, dst, c, 128, 128,
                             col_split=True)
    t = _tc_l2l3(a2a, a2b, W2, b2.reshape(1, -1), W3)
    p3a, p3b = _msgpass_call((t,), src, dst, c, 128, 128, col_split=False)
    return _tc_out(p3a[:, :64], p3b[:, :64], b3.reshape(1, -1))
